# baseline (speedup 1.0000x reference)
import jax
import jax.numpy as jnp
from jax.experimental import pallas as pl

D_MODEL = 128
NUM_COLORS = 10
COLORS_PAD = 16
PIX_BLK = 7200


def _pe_add_kernel(x_ref, idx_ref, sp_ref, ch_ref, out_ref):
    idx = idx_ref[...]
    lanes = jax.lax.broadcasted_iota(jnp.int32, (PIX_BLK, COLORS_PAD), 1)
    onehot = (idx == lanes).astype(jnp.float32)
    chrom = jnp.dot(onehot, ch_ref[...],
                    preferred_element_type=jnp.float32)
    out_ref[...] = x_ref[...] + sp_ref[...] + chrom


def kernel(x, color_indices, spatial_pe, chromatic_pe):
    Bb, Hh, Ww, d = x.shape
    half = d // 2
    hw = Hh * Ww
    n_pix = Bb * hw

    xf = x.reshape(n_pix, d)
    idxf = color_indices.astype(jnp.int32).reshape(n_pix, 1)
    sp128 = jnp.concatenate(
        [spatial_pe[:Hh, :Ww, :].reshape(hw, half),
         jnp.zeros((hw, half), dtype=x.dtype)], axis=-1)
    sp_tiled = jnp.tile(sp128, (PIX_BLK // hw, 1))
    ch128 = jnp.zeros((COLORS_PAD, d), dtype=x.dtype)
    ch128 = ch128.at[:NUM_COLORS, half:].set(chromatic_pe)

    grid = (n_pix // PIX_BLK,)
    out = pl.pallas_call(
        _pe_add_kernel,
        grid=grid,
        in_specs=[
            pl.BlockSpec((PIX_BLK, d), lambda i: (i, 0)),
            pl.BlockSpec((PIX_BLK, 1), lambda i: (i, 0)),
            pl.BlockSpec((PIX_BLK, d), lambda i: (0, 0)),
            pl.BlockSpec((COLORS_PAD, d), lambda i: (0, 0)),
        ],
        out_specs=pl.BlockSpec((PIX_BLK, d), lambda i: (i, 0)),
        out_shape=jax.ShapeDtypeStruct((n_pix, d), x.dtype),
    )(xf, idxf, sp_tiled, ch128)
    return out.reshape(Bb, Hh, Ww, d)


# speedup vs baseline: 2.1534x; 2.1534x over previous
import jax
import jax.numpy as jnp
from jax.experimental import pallas as pl
from jax.experimental.pallas import tpu as pltpu

D_MODEL = 128
NUM_COLORS = 10
B_BLK = 8


def _pe_add_kernel(x_ref, idx_ref, sp_ref, ch_ref, out_ref):
    x = x_ref[...]
    idx = idx_ref[...]
    sp = sp_ref[...]
    ch = ch_ref[...]

    idx3 = idx[..., None]
    chrom = jnp.broadcast_to(ch[0], x.shape)
    for c in range(1, NUM_COLORS):
        chrom = jnp.where(idx3 == c, ch[c], chrom)

    out_ref[...] = x + sp[None, :, :] + chrom


def kernel(x, color_indices, spatial_pe, chromatic_pe):
    Bb, Hh, Ww, d = x.shape
    half = d // 2
    hw = Hh * Ww

    xf = x.reshape(Bb, hw, d)
    idxf = color_indices.astype(jnp.int32).reshape(Bb, hw)
    zeros_half = jnp.zeros((hw, half), dtype=x.dtype)
    sp128 = jnp.concatenate(
        [spatial_pe[:Hh, :Ww, :].reshape(hw, half), zeros_half], axis=-1)
    ch128 = jnp.concatenate(
        [jnp.zeros((NUM_COLORS, half), dtype=x.dtype), chromatic_pe], axis=-1)

    grid = (Bb // B_BLK,)
    out = pl.pallas_call(
        _pe_add_kernel,
        grid=grid,
        in_specs=[
            pl.BlockSpec((B_BLK, hw, d), lambda i: (i, 0, 0)),
            pl.BlockSpec((B_BLK, hw), lambda i: (i, 0)),
            pl.BlockSpec((hw, d), lambda i: (0, 0)),
            pl.BlockSpec((NUM_COLORS, d), lambda i: (0, 0)),
        ],
        out_specs=pl.BlockSpec((B_BLK, hw, d), lambda i: (i, 0, 0)),
        out_shape=jax.ShapeDtypeStruct((Bb, hw, d), x.dtype),
        compiler_params=pltpu.CompilerParams(
            dimension_semantics=("parallel",)),
    )(xf, idxf, sp128, ch128)
    return out.reshape(Bb, Hh, Ww, d)
